# baseline (speedup 1.0000x reference)
import functools

import jax
import jax.numpy as jnp
from jax import lax
from jax.experimental import pallas as pl
from jax.experimental.pallas import tpu as pltpu, tpu_sc as plsc

DIM_VOCAB = 1000000
DIM_HIDDEN = 64
BATCH = 4096
HIST_LEN = 200

NUM_CORES = 2
NUM_SUBCORES = 16
NUM_WORKERS = NUM_CORES * NUM_SUBCORES

TOTAL = BATCH * HIST_LEN
PER_WORKER = TOTAL // NUM_WORKERS
CHUNK = 800
NUM_CHUNKS = PER_WORKER // CHUNK


def _embed_body(x_hbm, table_hbm, out_hbm, idx_v, rows_v, sem):
    wid = lax.axis_index("s") * NUM_CORES + lax.axis_index("c")
    base = wid * PER_WORKER

    def chunk_step(i, carry):
        off = base + i * CHUNK
        pltpu.sync_copy(x_hbm.at[pl.ds(off, CHUNK)], idx_v)
        pltpu.async_copy(table_hbm.at[idx_v], rows_v, sem).wait()
        pltpu.sync_copy(rows_v, out_hbm.at[pl.ds(off, CHUNK)])
        return carry

    lax.fori_loop(0, NUM_CHUNKS, chunk_step, 0)


@jax.jit
def _embed(x_flat, emb_t):
    mesh = plsc.VectorSubcoreMesh(
        core_axis_name="c", subcore_axis_name="s",
        num_cores=NUM_CORES, num_subcores=NUM_SUBCORES)
    run = functools.partial(
        pl.kernel,
        mesh=mesh,
        compiler_params=pltpu.CompilerParams(use_tc_tiling_on_sc=False),
        out_type=jax.ShapeDtypeStruct((TOTAL, DIM_HIDDEN), jnp.float32),
        scratch_types=[
            pltpu.VMEM((CHUNK,), jnp.int32),
            pltpu.VMEM((CHUNK, DIM_HIDDEN), jnp.float32),
            pltpu.SemaphoreType.DMA,
        ],
    )(_embed_body)
    return run(x_flat, emb_t)


def kernel(x, emb_t):
    x_flat = x.reshape(-1).astype(jnp.int32)
    y = _embed(x_flat, emb_t)
    return y.reshape(BATCH, HIST_LEN, DIM_HIDDEN)


# speedup vs baseline: 1.0175x; 1.0175x over previous
import functools

import jax
import jax.numpy as jnp
from jax import lax
from jax.experimental import pallas as pl
from jax.experimental.pallas import tpu as pltpu, tpu_sc as plsc

DIM_VOCAB = 1000000
DIM_HIDDEN = 64
BATCH = 4096
HIST_LEN = 200

NUM_CORES = 2
NUM_SUBCORES = 16
NUM_WORKERS = NUM_CORES * NUM_SUBCORES

TOTAL = BATCH * HIST_LEN
PER_WORKER = TOTAL // NUM_WORKERS
CHUNK = 800
NUM_CHUNKS = PER_WORKER // CHUNK
NBUF = 2


def _embed_body(x_hbm, table_hbm, out_hbm,
                idx0, idx1, rows0, rows1, g0, g1, o0, o1):
    idx = (idx0, idx1)
    rows = (rows0, rows1)
    gsem = (g0, g1)
    osem = (o0, o1)
    wid = lax.axis_index("s") * NUM_CORES + lax.axis_index("c")
    base = wid * PER_WORKER

    for b in range(NBUF):
        off = base + b * CHUNK
        pltpu.sync_copy(x_hbm.at[pl.ds(off, CHUNK)], idx[b])
        pltpu.async_copy(table_hbm.at[idx[b]], rows[b], gsem[b])

    def step(g, carry):
        for b in range(NBUF):
            c = NBUF * g + b
            off = base + c * CHUNK
            pltpu.make_async_copy(table_hbm.at[idx[b]], rows[b],
                                  gsem[b]).wait()
            pltpu.async_copy(rows[b], out_hbm.at[pl.ds(off, CHUNK)], osem[b])

            @pl.when(c + NBUF < NUM_CHUNKS)
            def _():
                pltpu.make_async_copy(
                    rows[b], out_hbm.at[pl.ds(off, CHUNK)], osem[b]).wait()
                off2 = off + NBUF * CHUNK
                pltpu.sync_copy(x_hbm.at[pl.ds(off2, CHUNK)], idx[b])
                pltpu.async_copy(table_hbm.at[idx[b]], rows[b], gsem[b])
        return carry

    lax.fori_loop(0, NUM_CHUNKS // NBUF, step, 0)

    for b in range(NBUF):
        off = base + (NUM_CHUNKS - NBUF + b) * CHUNK
        pltpu.make_async_copy(rows[b], out_hbm.at[pl.ds(off, CHUNK)],
                              osem[b]).wait()


@jax.jit
def _embed(x_flat, emb_t):
    mesh = plsc.VectorSubcoreMesh(
        core_axis_name="c", subcore_axis_name="s",
        num_cores=NUM_CORES, num_subcores=NUM_SUBCORES)
    run = functools.partial(
        pl.kernel,
        mesh=mesh,
        compiler_params=pltpu.CompilerParams(use_tc_tiling_on_sc=False),
        out_type=jax.ShapeDtypeStruct((TOTAL, DIM_HIDDEN), jnp.float32),
        scratch_types=[
            pltpu.VMEM((CHUNK,), jnp.int32),
            pltpu.VMEM((CHUNK,), jnp.int32),
            pltpu.VMEM((CHUNK, DIM_HIDDEN), jnp.float32),
            pltpu.VMEM((CHUNK, DIM_HIDDEN), jnp.float32),
            pltpu.SemaphoreType.DMA,
            pltpu.SemaphoreType.DMA,
            pltpu.SemaphoreType.DMA,
            pltpu.SemaphoreType.DMA,
        ],
    )(_embed_body)
    return run(x_flat, emb_t)


def kernel(x, emb_t):
    x_flat = x.reshape(-1).astype(jnp.int32)
    y = _embed(x_flat, emb_t)
    return y.reshape(BATCH, HIST_LEN, DIM_HIDDEN)
